# baseline (speedup 1.0000x reference)
import functools

import jax
import jax.numpy as jnp
from jax import lax
from jax.experimental import pallas as pl
from jax.experimental.pallas import tpu as pltpu
from jax.experimental.pallas import tpu_sc as plsc

NUM_CLASSES = 100000
HIDDEN_SIZE = 128
BATCH = 16384

_INFO = plsc.get_sparse_core_info()
_NC = _INFO.num_cores
_NS = _INFO.num_subcores
_NW = _NC * _NS
_B_PER_W = BATCH // _NW
_CH = 128
_NCH = _B_PER_W // _CH
_CHUNKS = (_CH,) * _NCH
_OFFS = tuple(i * _CH for i in range(_NCH))


@functools.partial(
    pl.kernel,
    out_type=jax.ShapeDtypeStruct((BATCH, HIDDEN_SIZE), jnp.float32),
    mesh=plsc.VectorSubcoreMesh(core_axis_name="c", subcore_axis_name="s"),
    scratch_types=[
        pltpu.VMEM((_B_PER_W,), jnp.int32),
        pltpu.VMEM((_B_PER_W, HIDDEN_SIZE), jnp.float32),
        pltpu.SemaphoreType.DMA,
        pltpu.SemaphoreType.DMA,
        pltpu.SemaphoreType.DMA,
    ],
)
def _gather_kernel(table_hbm, idx_hbm, out_hbm, idx_v, rows_v, i_sem, g_sem, w_sem):
    parity = lax.axis_index("s") % 2

    @pl.when(parity == 0)
    def _work():
        aw = lax.axis_index("c") * (_NS // 2) + lax.axis_index("s") // 2
        for r in range(2):
            rbase = (aw * 2 + r) * _B_PER_W
            pltpu.async_copy(idx_hbm.at[aw * 2 + r], idx_v, i_sem).wait()
            gathers = [
                pltpu.async_copy(
                    table_hbm.at[idx_v.at[pl.ds(o, n)]],
                    rows_v.at[pl.ds(o, n)],
                    g_sem,
                )
                for o, n in zip(_OFFS, _CHUNKS)
            ]
            for g in gathers:
                g.wait()
            pltpu.async_copy(rows_v, out_hbm.at[pl.ds(rbase, _B_PER_W)], w_sem).wait()


def kernel(labels, table):
    idx = labels.astype(jnp.int32).reshape(_NW, _B_PER_W)
    return _gather_kernel(table, idx)


# speedup vs baseline: 1.2183x; 1.2183x over previous
import functools

import jax
import jax.numpy as jnp
from jax import lax
from jax.experimental import pallas as pl
from jax.experimental.pallas import tpu as pltpu
from jax.experimental.pallas import tpu_sc as plsc

NUM_CLASSES = 100000
HIDDEN_SIZE = 128
BATCH = 16384

_INFO = plsc.get_sparse_core_info()
_NC = _INFO.num_cores
_NS = _INFO.num_subcores
_NW = _NC * _NS
_B_PER_W = BATCH // _NW
_CH = 128
_NCH = _B_PER_W // _CH
_CHUNKS = (_CH,) * _NCH
_OFFS = tuple(i * _CH for i in range(_NCH))


@functools.partial(
    pl.kernel,
    out_type=jax.ShapeDtypeStruct((BATCH, HIDDEN_SIZE), jnp.float32),
    mesh=plsc.VectorSubcoreMesh(core_axis_name="c", subcore_axis_name="s"),
    scratch_types=[
        pltpu.VMEM((_B_PER_W,), jnp.int32),
        pltpu.VMEM((_B_PER_W, HIDDEN_SIZE), jnp.float32),
        pltpu.SemaphoreType.DMA,
        pltpu.SemaphoreType.DMA,
        pltpu.SemaphoreType.DMA,
    ],
)
def _gather_kernel(table_hbm, idx_hbm, out_hbm, idx_v, rows_v, i_sem, g_sem, w_sem):
    wid = lax.axis_index("c") * _NS + lax.axis_index("s")
    base = wid * _B_PER_W
    pltpu.async_copy(idx_hbm.at[wid], idx_v, i_sem).wait()
    gathers = [
        pltpu.async_copy(
            table_hbm.at[idx_v.at[pl.ds(o, n)]],
            rows_v.at[pl.ds(o, n)],
            g_sem,
        )
        for o, n in zip(_OFFS, _CHUNKS)
    ]
    for g in gathers:
        g.wait()
    pltpu.async_copy(rows_v, out_hbm.at[pl.ds(base, _B_PER_W)], w_sem).wait()


def kernel(labels, table):
    idx = labels.astype(jnp.int32).reshape(_NW, _B_PER_W)
    return _gather_kernel(table, idx)
